# baseline (speedup 1.0000x reference)
import jax
import jax.numpy as jnp
from jax.experimental import pallas as pl
from jax.experimental.pallas import tpu as pltpu

_NUM_FEATURES = 256
_NUM_BATCHES = 8
_N_POINTS = 65536
_EPS = 1e-05
_R = 8192
_NBLK = _N_POINTS // _R
_NSPLIT = 4
_RS = _R // _NSPLIT


def _onehot(idx_ref, h):
    idx = idx_ref[0, :, pl.ds(h * _RS, _RS)]
    iota = jax.lax.broadcasted_iota(jnp.int32, (_NUM_BATCHES, _RS), 0)
    return (iota == idx).astype(jnp.float32)


def _fused_body(xa_ref, xb_ref, xc_ref, xd_ref, idx_ref, w_ref, b_ref, mw_ref, vw_ref, o_ref,
                sums_ref, sumsq_ref, cnt_ref, scale_ref, shift_ref):
    i = pl.program_id(0)

    @pl.when(i < _NBLK)
    def _phase1():
        s = jnp.zeros((_NUM_BATCHES, _NUM_FEATURES), jnp.float32)
        sq = jnp.zeros((_NUM_BATCHES, _NUM_FEATURES), jnp.float32)
        c = jnp.zeros((_NUM_BATCHES, 1), jnp.float32)
        dn = (((1,), (0,)), ((), ()))
        for h, ref in ((0, xa_ref), (1, xb_ref), (2, xc_ref), (3, xd_ref)):
            xh = ref[...]
            onehot = _onehot(idx_ref, h)
            s += jax.lax.dot_general(onehot, xh, dn,
                                     preferred_element_type=jnp.float32)
            sq += jax.lax.dot_general(onehot, xh * xh, dn,
                                      preferred_element_type=jnp.float32)
            c += jnp.sum(onehot, axis=1, keepdims=True)
        cb = jnp.broadcast_to(c, (_NUM_BATCHES, 128))

        @pl.when(i == 0)
        def _init():
            sums_ref[...] = s
            sumsq_ref[...] = sq
            cnt_ref[...] = cb

        @pl.when(i != 0)
        def _acc():
            sums_ref[...] += s
            sumsq_ref[...] += sq
            cnt_ref[...] += cb

        @pl.when(i == _NBLK - 1)
        def _finalize():
            cnt = cnt_ref[:, 0:1]
            cs = jnp.maximum(cnt, 1.0)
            sums = sums_ref[...]
            sumsq = sumsq_ref[...]
            mean_in = sums / cs
            ex2 = sumsq / cs
            var_in = ex2 - mean_in * mean_in
            mean_ln = jnp.mean(mean_in, axis=1, keepdims=True)
            var_ln = jnp.mean(ex2, axis=1, keepdims=True) - mean_ln * mean_ln
            tot_s = jnp.sum(sums, axis=0, keepdims=True)
            tot_sq = jnp.sum(sumsq, axis=0, keepdims=True)
            n = jnp.float32(_N_POINTS)
            mean_bn = tot_s / n
            var_bn = (tot_sq - n * mean_bn * mean_bn) / (n - 1.0)

            mw = mw_ref[...]
            mw = jnp.exp(mw - jnp.max(mw, axis=1, keepdims=True))
            mw = mw / jnp.sum(mw, axis=1, keepdims=True)
            vw = vw_ref[...]
            vw = jnp.exp(vw - jnp.max(vw, axis=1, keepdims=True))
            vw = vw / jnp.sum(vw, axis=1, keepdims=True)

            mean = (mw[:, 0:1] * mean_in + mw[:, 1:2] * mean_ln
                    + mw[:, 2:3] * mean_bn)
            var = (vw[:, 0:1] * var_in + vw[:, 1:2] * var_ln
                   + vw[:, 2:3] * var_bn)
            inv = jax.lax.rsqrt(var + _EPS)
            scale_ref[...] = inv * w_ref[...]
            shift_ref[...] = b_ref[...] - mean * (inv * w_ref[...])

    @pl.when(i >= _NBLK)
    def _phase2():
        dn = (((0,), (0,)), ((), ()))
        for h, ref in ((0, xa_ref), (1, xb_ref), (2, xc_ref), (3, xd_ref)):
            onehot = _onehot(idx_ref, h)
            g_scale = jax.lax.dot_general(onehot, scale_ref[...], dn,
                                          preferred_element_type=jnp.float32)
            g_shift = jax.lax.dot_general(onehot, shift_ref[...], dn,
                                          preferred_element_type=jnp.float32)
            o_ref[pl.ds(h * _RS, _RS), :] = ref[...] * g_scale + g_shift


def _blk(i):
    return jnp.where(i <= _NBLK, jnp.where(i < _NBLK, i, _NBLK - 1),
                     i - (_NBLK + 1))


def _x_imap(h):
    return lambda i: (_NSPLIT * _blk(i) + h, 0)


def _idx_imap(i):
    return (_blk(i), 0, 0)


def _out_imap(i):
    return (jnp.where(i <= _NBLK, _NBLK - 1, i - (_NBLK + 1)), 0)


def kernel(x, weight, bias, mean_weight, var_weight, batch_indices):
    idx3 = batch_indices.reshape(_NBLK, 1, _R)
    mw2 = mean_weight.reshape(1, 3)
    vw2 = var_weight.reshape(1, 3)

    full = lambda shape: pl.BlockSpec(shape, lambda i: tuple(0 for _ in shape))

    out = pl.pallas_call(
        _fused_body,
        grid=(2 * _NBLK,),
        in_specs=[
            pl.BlockSpec((_RS, _NUM_FEATURES), _x_imap(0)),
            pl.BlockSpec((_RS, _NUM_FEATURES), _x_imap(1)),
            pl.BlockSpec((_RS, _NUM_FEATURES), _x_imap(2)),
            pl.BlockSpec((_RS, _NUM_FEATURES), _x_imap(3)),
            pl.BlockSpec((1, 1, _R), _idx_imap),
            full((1, _NUM_FEATURES)), full((1, _NUM_FEATURES)),
            full((1, 3)), full((1, 3)),
        ],
        out_specs=pl.BlockSpec((_R, _NUM_FEATURES), _out_imap),
        out_shape=jax.ShapeDtypeStruct((_N_POINTS, _NUM_FEATURES),
                                       jnp.float32),
        compiler_params=pltpu.CompilerParams(
            vmem_limit_bytes=100 * 1024 * 1024),
        scratch_shapes=[
            pltpu.VMEM((_NUM_BATCHES, _NUM_FEATURES), jnp.float32),
            pltpu.VMEM((_NUM_BATCHES, _NUM_FEATURES), jnp.float32),
            pltpu.VMEM((_NUM_BATCHES, 128), jnp.float32),
            pltpu.VMEM((_NUM_BATCHES, _NUM_FEATURES), jnp.float32),
            pltpu.VMEM((_NUM_BATCHES, _NUM_FEATURES), jnp.float32),
        ],
    )(x, x, x, x, idx3, weight, bias, mw2, vw2)
    return out


# speedup vs baseline: 1.0046x; 1.0046x over previous
import jax
import jax.numpy as jnp
from jax.experimental import pallas as pl
from jax.experimental.pallas import tpu as pltpu

_NUM_FEATURES = 256
_NUM_BATCHES = 8
_N_POINTS = 65536
_EPS = 1e-05
_R = 8192
_NBLK = _N_POINTS // _R
_NSPLIT = 2
_RS = _R // _NSPLIT


def _onehot(idx_ref, h):
    idx = idx_ref[0, :, pl.ds(h * _RS, _RS)]
    iota = jax.lax.broadcasted_iota(jnp.int32, (_NUM_BATCHES, _RS), 0)
    return (iota == idx).astype(jnp.float32)


def _fused_body(xa_ref, xb_ref, idx_ref, w_ref, b_ref, mw_ref, vw_ref,
                o_ref,
                sums_ref, sumsq_ref, cnt_ref, scale_ref, shift_ref):
    i = pl.program_id(0)

    @pl.when(i < _NBLK)
    def _phase1():
        s = jnp.zeros((_NUM_BATCHES, _NUM_FEATURES), jnp.float32)
        sq = jnp.zeros((_NUM_BATCHES, _NUM_FEATURES), jnp.float32)
        c = jnp.zeros((_NUM_BATCHES, 1), jnp.float32)
        dn = (((1,), (0,)), ((), ()))
        for h, ref in ((0, xa_ref), (1, xb_ref)):
            xh = ref[...]
            onehot = _onehot(idx_ref, h)
            s += jax.lax.dot_general(onehot, xh, dn,
                                     preferred_element_type=jnp.float32)
            sq += jax.lax.dot_general(onehot, xh * xh, dn,
                                      preferred_element_type=jnp.float32)
            c += jnp.sum(onehot, axis=1, keepdims=True)
        cb = jnp.broadcast_to(c, (_NUM_BATCHES, 128))

        @pl.when(i == 0)
        def _init():
            sums_ref[...] = s
            sumsq_ref[...] = sq
            cnt_ref[...] = cb

        @pl.when(i != 0)
        def _acc():
            sums_ref[...] += s
            sumsq_ref[...] += sq
            cnt_ref[...] += cb

        @pl.when(i == _NBLK - 1)
        def _finalize():
            cnt = cnt_ref[:, 0:1]
            cs = jnp.maximum(cnt, 1.0)
            sums = sums_ref[...]
            sumsq = sumsq_ref[...]
            mean_in = sums / cs
            ex2 = sumsq / cs
            var_in = ex2 - mean_in * mean_in
            mean_ln = jnp.mean(mean_in, axis=1, keepdims=True)
            var_ln = jnp.mean(ex2, axis=1, keepdims=True) - mean_ln * mean_ln
            tot_s = jnp.sum(sums, axis=0, keepdims=True)
            tot_sq = jnp.sum(sumsq, axis=0, keepdims=True)
            n = jnp.float32(_N_POINTS)
            mean_bn = tot_s / n
            var_bn = (tot_sq - n * mean_bn * mean_bn) / (n - 1.0)

            mw = mw_ref[...]
            mw = jnp.exp(mw - jnp.max(mw, axis=1, keepdims=True))
            mw = mw / jnp.sum(mw, axis=1, keepdims=True)
            vw = vw_ref[...]
            vw = jnp.exp(vw - jnp.max(vw, axis=1, keepdims=True))
            vw = vw / jnp.sum(vw, axis=1, keepdims=True)

            mean = (mw[:, 0:1] * mean_in + mw[:, 1:2] * mean_ln
                    + mw[:, 2:3] * mean_bn)
            var = (vw[:, 0:1] * var_in + vw[:, 1:2] * var_ln
                   + vw[:, 2:3] * var_bn)
            inv = jax.lax.rsqrt(var + _EPS)
            scale_ref[...] = inv * w_ref[...]
            shift_ref[...] = b_ref[...] - mean * (inv * w_ref[...])

    @pl.when(i >= _NBLK)
    def _phase2():
        dn = (((0,), (0,)), ((), ()))
        for h, ref in ((0, xa_ref), (1, xb_ref)):
            onehot = _onehot(idx_ref, h)
            g_scale = jax.lax.dot_general(onehot, scale_ref[...], dn,
                                          preferred_element_type=jnp.float32)
            g_shift = jax.lax.dot_general(onehot, shift_ref[...], dn,
                                          preferred_element_type=jnp.float32)
            o_ref[pl.ds(h * _RS, _RS), :] = ref[...] * g_scale + g_shift


def _blk(i):
    return jnp.where(i <= _NBLK, jnp.where(i < _NBLK, i, _NBLK - 1),
                     i - (_NBLK + 1))


def _x_imap(h):
    return lambda i: (_NSPLIT * _blk(i) + h, 0)


def _idx_imap(i):
    return (_blk(i), 0, 0)


def _out_imap(i):
    return (jnp.where(i <= _NBLK, _NBLK - 1, i - (_NBLK + 1)), 0)


def kernel(x, weight, bias, mean_weight, var_weight, batch_indices):
    idx3 = batch_indices.reshape(_NBLK, 1, _R)
    mw2 = mean_weight.reshape(1, 3)
    vw2 = var_weight.reshape(1, 3)

    full = lambda shape: pl.BlockSpec(shape, lambda i: tuple(0 for _ in shape))

    out = pl.pallas_call(
        _fused_body,
        grid=(2 * _NBLK,),
        in_specs=[
            pl.BlockSpec((_RS, _NUM_FEATURES), _x_imap(0)),
            pl.BlockSpec((_RS, _NUM_FEATURES), _x_imap(1)),
            pl.BlockSpec((1, 1, _R), _idx_imap),
            full((1, _NUM_FEATURES)), full((1, _NUM_FEATURES)),
            full((1, 3)), full((1, 3)),
        ],
        out_specs=pl.BlockSpec((_R, _NUM_FEATURES), _out_imap),
        out_shape=jax.ShapeDtypeStruct((_N_POINTS, _NUM_FEATURES),
                                       jnp.float32),
        compiler_params=pltpu.CompilerParams(
            vmem_limit_bytes=100 * 1024 * 1024),
        scratch_shapes=[
            pltpu.VMEM((_NUM_BATCHES, _NUM_FEATURES), jnp.float32),
            pltpu.VMEM((_NUM_BATCHES, _NUM_FEATURES), jnp.float32),
            pltpu.VMEM((_NUM_BATCHES, 128), jnp.float32),
            pltpu.VMEM((_NUM_BATCHES, _NUM_FEATURES), jnp.float32),
            pltpu.VMEM((_NUM_BATCHES, _NUM_FEATURES), jnp.float32),
        ],
    )(x, x, idx3, weight, bias, mw2, vw2)
    return out
